# baseline (speedup 1.0000x reference)
import functools

import jax
import jax.numpy as jnp
from jax.experimental import pallas as pl
from jax.experimental.pallas import tpu as pltpu


def _band_kernel(cur_ref, w_ref, b_ref, out_ref, xprev_ref, uprow_ref, *, W, HB):
    NB = HB * W
    h = pl.program_id(1)
    nsteps = pl.num_programs(1)

    @pl.when(h > 0)
    def _compute_band():
        xb = xprev_ref[...]
        up_row = jnp.where(h == 1, xb[:, :W], uprow_ref[...])
        down_row = jnp.where(h == nsteps - 1, xb[:, NB - W :], cur_ref[0, :, :W])

        up = jnp.concatenate([up_row, xb[:, : NB - W]], axis=1)
        down = jnp.concatenate([xb[:, W:], down_row], axis=1)
        colsum = up + xb + down

        wpos = jax.lax.broadcasted_iota(jnp.int32, (1, NB), 1) % W
        left = jnp.concatenate([colsum[:, :1], colsum[:, :-1]], axis=1)
        left = jnp.where(wpos == 0, colsum, left)
        right = jnp.concatenate([colsum[:, 1:], colsum[:, -1:]], axis=1)
        right = jnp.where(wpos == W - 1, colsum, right)
        mean = (left + colsum + right - xb) * 0.125

        agg = jnp.concatenate([xb, mean], axis=0)
        y = jnp.dot(w_ref[...], agg, preferred_element_type=jnp.float32)
        out_ref[0] = jnp.maximum(y + b_ref[...], 0.0) + xb

    uprow_ref[...] = xprev_ref[:, NB - W :]
    xprev_ref[...] = cur_ref[0]


def kernel(x, W_proj, b_proj):
    B, C, H, W = x.shape
    N = H * W
    HB = 28
    nbands = H // HB
    NB = HB * W

    x2 = x.reshape(B, C, N)
    b2 = b_proj.reshape(C, 1)

    grid = (B, nbands + 1)
    out = pl.pallas_call(
        functools.partial(_band_kernel, W=W, HB=HB),
        grid=grid,
        in_specs=[
            pl.BlockSpec(
                (1, C, NB),
                lambda b, h: (b, 0, jnp.minimum(h, nbands - 1)),
            ),
            pl.BlockSpec((C, 2 * C), lambda b, h: (0, 0)),
            pl.BlockSpec((C, 1), lambda b, h: (0, 0)),
        ],
        out_specs=pl.BlockSpec(
            (1, C, NB),
            lambda b, h: (b, 0, (h + nbands - 1) % nbands),
        ),
        out_shape=jax.ShapeDtypeStruct((B, C, N), jnp.float32),
        scratch_shapes=[
            pltpu.VMEM((C, NB), jnp.float32),
            pltpu.VMEM((C, W), jnp.float32),
        ],
        compiler_params=pltpu.CompilerParams(
            dimension_semantics=("parallel", "arbitrary"),
        ),
    )(x2, W_proj, b2)
    return out.reshape(B, C, H, W)


# speedup vs baseline: 1.0352x; 1.0352x over previous
import functools

import jax
import jax.numpy as jnp
from jax.experimental import pallas as pl
from jax.experimental.pallas import tpu as pltpu


def _band_kernel(cur_ref, up_ref, down_ref, w_ref, b_ref, out_ref, *, W, HB):
    C = cur_ref.shape[1]
    NB = HB * W
    h = pl.program_id(1)
    nbands = pl.num_programs(1)
    xb = cur_ref[0]
    w1p = w_ref[:C]
    w2p = w_ref[C:]

    up_blk = up_ref[0]
    down_blk = down_ref[0]
    up_row = jnp.where(h == 0, up_blk[:, :W], up_blk[:, 3 * W:])
    down_row = jnp.where(h == nbands - 1, down_blk[:, 3 * W:], down_blk[:, :W])

    p = jnp.dot(w2p, xb, preferred_element_type=jnp.float32)
    p_up = jnp.dot(w2p, up_row, preferred_element_type=jnp.float32)
    p_down = jnp.dot(w2p, down_row, preferred_element_type=jnp.float32)

    up = jnp.concatenate([p_up, p[:, : NB - W]], axis=1)
    down = jnp.concatenate([p[:, W:], p_down], axis=1)
    colsum = up + p + down

    wpos = jax.lax.broadcasted_iota(jnp.int32, (1, NB), 1) % W
    left = jnp.concatenate([colsum[:, :1], colsum[:, :-1]], axis=1)
    left = jnp.where(wpos == 0, colsum, left)
    right = jnp.concatenate([colsum[:, 1:], colsum[:, -1:]], axis=1)
    right = jnp.where(wpos == W - 1, colsum, right)

    y = jnp.dot(w1p, xb, preferred_element_type=jnp.float32)
    y = y + (left + colsum + right) + b_ref[...]
    out_ref[0] = jnp.maximum(y, 0.0) + xb


def kernel(x, W_proj, b_proj):
    B, C, H, W = x.shape
    N = H * W
    HB = 28
    nbands = H // HB
    NB = HB * W
    RPB = HB // 4

    x2 = x.reshape(B, C, N)
    b2 = b_proj.reshape(C, 1)
    w1 = W_proj[:, :C]
    w2 = W_proj[:, C:]
    wf = jnp.concatenate([w1 - 0.125 * w2, 0.125 * w2], axis=0)

    grid = (B, nbands)
    out = pl.pallas_call(
        functools.partial(_band_kernel, W=W, HB=HB),
        grid=grid,
        in_specs=[
            pl.BlockSpec((1, C, NB), lambda b, h: (b, 0, h)),
            pl.BlockSpec(
                (1, C, 4 * W),
                lambda b, h: (b, 0, jnp.maximum(h * RPB - 1, 0)),
            ),
            pl.BlockSpec(
                (1, C, 4 * W),
                lambda b, h: (b, 0, jnp.minimum((h + 1) * RPB, nbands * RPB - 1)),
            ),
            pl.BlockSpec((2 * C, C), lambda b, h: (0, 0)),
            pl.BlockSpec((C, 1), lambda b, h: (0, 0)),
        ],
        out_specs=pl.BlockSpec((1, C, NB), lambda b, h: (b, 0, h)),
        out_shape=jax.ShapeDtypeStruct((B, C, N), jnp.float32),
        compiler_params=pltpu.CompilerParams(
            dimension_semantics=("parallel", "arbitrary"),
        ),
    )(x2, x2, x2, wf, b2)
    return out.reshape(B, C, H, W)
